# baseline (speedup 1.0000x reference)
import functools

import jax
import jax.numpy as jnp
from jax.experimental import pallas as pl
from jax.experimental.pallas import tpu as pltpu

B = 32
F = 64
BM = 4096
G = 8


def _score_copy_body(x_ref, w1a_ref, w1bt_ref, b1_ref, w2blk_ref, thr_ref,
                     mem_ref, out_mem_ref, patch_ref, mask_ref, groups_ref,
                     m_s, s_s, idx_s):
    i = pl.program_id(0)
    nblk = pl.num_programs(0)

    @pl.when(i == 0)
    def _init():
        m_s[...] = jnp.full((B, 128), -jnp.inf, jnp.float32)
        s_s[...] = jnp.zeros((B, 128), jnp.float32)
        idx_s[...] = jnp.zeros((B, 128), jnp.int32)

    x = x_ref[...]
    in_proj = jnp.dot(x, w1a_ref[...],
                      preferred_element_type=jnp.float32) + b1_ref[...]
    memb = mem_ref[...]
    out_mem_ref[...] = memb
    mem_projT = jax.lax.dot_general(
        w1bt_ref[...], memb, (((1,), (1,)), ((), ())),
        preferred_element_type=jnp.float32)

    w2blk = w2blk_ref[...]
    tiles = []
    for g in range(B // G):
        ipg = in_proj[g * G:(g + 1) * G, :]
        slab = jnp.tanh(mem_projT[None, :, :]
                        + ipg[:, :, None]).reshape(G * F, BM)
        tiles.append(jnp.dot(w2blk, slab,
                             preferred_element_type=jnp.float32))
    scores = jnp.concatenate(tiles, axis=0)

    lane_idx = jax.lax.broadcasted_iota(jnp.int32, (B, 128), 1)
    ntile = BM // 128
    tiles_s = [scores[:, t * 128:(t + 1) * 128] for t in range(ntile)]
    m_old = m_s[...]
    m_new = m_old
    for t in range(ntile):
        m_new = jnp.maximum(m_new, tiles_s[t])
    s_acc = jnp.exp(tiles_s[0] - m_new)
    for t in range(1, ntile):
        s_acc = s_acc + jnp.exp(tiles_s[t] - m_new)
    s_s[...] = s_s[...] * jnp.exp(m_old - m_new) + s_acc
    improved = m_new > m_old
    idx = idx_s[...]
    for t in reversed(range(ntile)):
        idx = jnp.where(improved & (tiles_s[t] == m_new),
                        i * BM + t * 128 + lane_idx, idx)
    idx_s[...] = idx
    m_s[...] = m_new

    @pl.when(i == nblk - 1)
    def _finalize():
        m_lane = m_s[...]
        m_g = jnp.max(m_lane, axis=1, keepdims=True)
        s = jnp.sum(s_s[...] * jnp.exp(m_lane - m_g),
                    axis=1, keepdims=True)
        cand = jnp.where(m_lane == m_g, idx_s[...], jnp.int32(2 ** 30))
        slot = jnp.min(cand, axis=1, keepdims=True)
        best_w = 1.0 / s
        do_write = best_w > thr_ref[...]
        eq = slot == slot.reshape(1, B)
        ii = jax.lax.broadcasted_iota(jnp.int32, (B, B), 0)
        jj = jax.lax.broadcasted_iota(jnp.int32, (B, B), 1)
        conflict = jnp.any(eq & (jj > ii) & do_write.reshape(1, B),
                           axis=1, keepdims=True)
        final_write = do_write & jnp.logical_not(conflict)
        group = slot // G
        row = slot % G
        r8 = jax.lax.broadcasted_iota(jnp.int32, (1, G, 1), 1)
        match3 = (final_write.reshape(1, 1, B)
                  & (group.reshape(1, 1, B) == group.reshape(B, 1, 1))
                  & (row.reshape(1, 1, B) == r8))
        match2 = jnp.where(match3, 1.0, 0.0).reshape(B * G, B)
        mask_ref[...] = jnp.sum(match2, axis=1, keepdims=True)
        patch_ref[...] = jnp.dot(match2, x,
                                 preferred_element_type=jnp.float32)
        groups_ref[...] = group.reshape(1, B)


def _scatter_body(groups_ref, patch_ref, mask_ref, cur_ref, out_ref):
    out_ref[...] = jnp.where(mask_ref[...] != 0.0,
                             patch_ref[...], cur_ref[...])


@functools.partial(jax.jit, static_argnames=())
def kernel(input_data, memory, W1, b1, W2, b2, threshold):
    del b2
    M = memory.shape[0]
    nblk = M // BM

    w1a = W1[:F, :]
    w1bt = W1[F:, :].T
    b1r = b1.reshape(1, F)
    thr = threshold.reshape(1, 1)
    w2row = W2.reshape(1, F)
    w2blk = jnp.zeros((G, G * F), jnp.float32)
    for g in range(G):
        w2blk = w2blk.at[g:g + 1, g * F:(g + 1) * F].set(w2row)

    out_mem, patch, mask, groups = pl.pallas_call(
        _score_copy_body,
        grid=(nblk,),
        in_specs=[
            pl.BlockSpec((B, F), lambda i: (0, 0)),
            pl.BlockSpec((F, F), lambda i: (0, 0)),
            pl.BlockSpec((F, F), lambda i: (0, 0)),
            pl.BlockSpec((1, F), lambda i: (0, 0)),
            pl.BlockSpec((G, G * F), lambda i: (0, 0)),
            pl.BlockSpec((1, 1), lambda i: (0, 0)),
            pl.BlockSpec((BM, F), lambda i: (i, 0)),
        ],
        out_specs=[
            pl.BlockSpec((BM, F), lambda i: (i, 0)),
            pl.BlockSpec((B * G, F), lambda i: (0, 0)),
            pl.BlockSpec((B * G, 1), lambda i: (0, 0)),
            pl.BlockSpec((1, B), lambda i: (0, 0)),
        ],
        out_shape=[
            jax.ShapeDtypeStruct((M, F), jnp.float32),
            jax.ShapeDtypeStruct((B * G, F), jnp.float32),
            jax.ShapeDtypeStruct((B * G, 1), jnp.float32),
            jax.ShapeDtypeStruct((1, B), jnp.int32),
        ],
        scratch_shapes=[
            pltpu.VMEM((B, 128), jnp.float32),
            pltpu.VMEM((B, 128), jnp.float32),
            pltpu.VMEM((B, 128), jnp.int32),
        ],
    )(input_data, w1a, w1bt, b1r, w2blk, thr, memory)

    groups1d = groups.reshape(B)

    grid_spec = pltpu.PrefetchScalarGridSpec(
        num_scalar_prefetch=1,
        grid=(B,),
        in_specs=[
            pl.BlockSpec((G, F), lambda i, g: (i, 0)),
            pl.BlockSpec((G, 1), lambda i, g: (i, 0)),
            pl.BlockSpec((G, F), lambda i, g: (g[i], 0)),
        ],
        out_specs=pl.BlockSpec((G, F), lambda i, g: (g[i], 0)),
    )
    updated = pl.pallas_call(
        _scatter_body,
        grid_spec=grid_spec,
        out_shape=jax.ShapeDtypeStruct((M, F), jnp.float32),
        input_output_aliases={3: 0},
    )(groups1d, patch, mask, out_mem)
    return updated
